# baseline (speedup 1.0000x reference)
import functools

import jax
import jax.numpy as jnp
from jax import lax
from jax.experimental import pallas as pl
from jax.experimental.pallas import tpu as pltpu
from jax.experimental.pallas import tpu_sc as plsc

KCB = 8192
D = 256
NTOK = 16384
BETA = 0.25

TM = 256
NT = NTOK // TM
KC = 1024
NKC = KCB // KC

NC, NS, L = 2, 16, 16
NW = NC * NS
BPW = NTOK // NW
CH = 128
NCHUNK = BPW // CH


def _dist_argmin_body(e_ref, c_ref, idx_ref):
    e = e_ref[...]
    a = jnp.sum(e * e, axis=1, keepdims=True)
    best = jnp.full((TM,), jnp.inf, dtype=jnp.float32)
    bidx = jnp.zeros((TM,), dtype=jnp.int32)
    iota = lax.broadcasted_iota(jnp.int32, (TM, KC), 1)
    for j in range(NKC):
        c = c_ref[pl.ds(j * KC, KC), :]
        cn = jnp.sum(c * c, axis=1)
        m = lax.dot_general(e, c, (((1,), (1,)), ((), ())),
                            preferred_element_type=jnp.float32)
        d2 = (a - 2.0 * m) + cn[None, :]
        tmin = jnp.min(d2, axis=1)
        tidx = jnp.min(jnp.where(d2 == tmin[:, None], iota, KCB), axis=1)
        upd = tmin < best
        bidx = jnp.where(upd, tidx + (j * KC), bidx)
        best = jnp.where(upd, tmin, best)
    idx_ref[0, 0, :] = bidx


def _dist_argmin(encoding, codebook):
    return pl.pallas_call(
        _dist_argmin_body,
        grid=(NT,),
        in_specs=[
            pl.BlockSpec((TM, D), lambda i: (i, 0)),
            pl.BlockSpec((KCB, D), lambda i: (0, 0)),
        ],
        out_specs=pl.BlockSpec((1, 1, TM), lambda i: (i, 0, 0)),
        out_shape=jax.ShapeDtypeStruct((NT, 1, TM), jnp.int32),
        compiler_params=pltpu.CompilerParams(
            dimension_semantics=("arbitrary",)),
    )(encoding, codebook)


def _sc_gather(codebook, idx):
    idx3 = idx.reshape(NW, NCHUNK, CH)
    mesh = plsc.VectorSubcoreMesh(core_axis_name="c", subcore_axis_name="s")

    @functools.partial(
        pl.kernel,
        out_type=jax.ShapeDtypeStruct((NTOK, D), jnp.float32),
        mesh=mesh,
        scratch_types=[
            pltpu.VMEM((NCHUNK, CH), jnp.int32),
            pltpu.VMEM((CH, D), jnp.float32),
            pltpu.VMEM((CH, D), jnp.float32),
            pltpu.SemaphoreType.DMA,
            pltpu.SemaphoreType.DMA,
        ],
    )
    def k(cb_hbm, idx_hbm, out_hbm, idx_v, buf0, buf1, sem0, sem1):
        wid = lax.axis_index("s") * NC + lax.axis_index("c")
        base = wid * BPW
        pltpu.sync_copy(idx_hbm.at[wid], idx_v)
        bufs = (buf0, buf1)
        sems = (sem0, sem1)
        cp = pltpu.async_copy(cb_hbm.at[idx_v.at[0]], bufs[0], sems[0])
        for ci in range(NCHUNK):
            nxt = None
            if ci + 1 < NCHUNK:
                nxt = pltpu.async_copy(
                    cb_hbm.at[idx_v.at[ci + 1]], bufs[(ci + 1) % 2],
                    sems[(ci + 1) % 2])
            cp.wait()
            pltpu.sync_copy(bufs[ci % 2], out_hbm.at[pl.ds(base + ci * CH, CH)])
            cp = nxt

    return k(codebook, idx3)


def _st_loss_body(e_ref, q_ref, qst_ref, part_ref):
    i = pl.program_id(0)
    e = e_ref[...]
    q = q_ref[...]
    d = q - e
    qst_ref[...] = e + d

    @pl.when(i == 0)
    def _():
        part_ref[...] = jnp.zeros_like(part_ref)

    part_ref[0, 0, :] = part_ref[0, 0, :] + jnp.sum(d * d, axis=0)


def _st_loss(encoding, q):
    return pl.pallas_call(
        _st_loss_body,
        grid=(NT,),
        in_specs=[
            pl.BlockSpec((TM, D), lambda i: (i, 0)),
            pl.BlockSpec((TM, D), lambda i: (i, 0)),
        ],
        out_specs=[
            pl.BlockSpec((TM, D), lambda i: (i, 0)),
            pl.BlockSpec((1, 1, D), lambda i: (0, 0, 0)),
        ],
        out_shape=[
            jax.ShapeDtypeStruct((NTOK, D), jnp.float32),
            jax.ShapeDtypeStruct((1, 1, D), jnp.float32),
        ],
        compiler_params=pltpu.CompilerParams(
            dimension_semantics=("arbitrary",)),
    )(encoding, q)


def kernel(encoding, codebook):
    idx = _dist_argmin(encoding, codebook).reshape(NTOK)
    q = _sc_gather(codebook, idx)
    qst, parts = _st_loss(encoding, q)
    mse = jnp.sum(parts) / (NTOK * D)
    commitment_loss = mse
    embedding_loss = mse
    vq_loss = commitment_loss * BETA + embedding_loss
    return (idx.reshape(NTOK, 1), qst, vq_loss, embedding_loss, commitment_loss)


# speedup vs baseline: 1.5046x; 1.5046x over previous
import functools

import jax
import jax.numpy as jnp
from jax import lax
from jax.experimental import pallas as pl
from jax.experimental.pallas import tpu as pltpu
from jax.experimental.pallas import tpu_sc as plsc

KCB = 8192
D = 256
NTOK = 16384
BETA = 0.25

TM = 256
NT = NTOK // TM
KC = 1024
NKC = KCB // KC

NC, NS, L = 2, 16, 16
NW = NC * NS
BPW = NTOK // NW
CH = 128
NCHUNK = BPW // CH


def _dist_argmin_body(e_ref, c_ref, idx_ref, cn_ref):
    i = pl.program_id(0)

    @pl.when(i == 0)
    def _():
        for j in range(NKC):
            c = c_ref[pl.ds(j * KC, KC), :]
            cn_ref[0, pl.ds(j * KC, KC)] = jnp.sum(c * c, axis=1)

    e = e_ref[...]
    a = jnp.sum(e * e, axis=1, keepdims=True)
    best = jnp.full((TM,), jnp.inf, dtype=jnp.float32)
    bidx = jnp.full((TM,), float(KCB), dtype=jnp.float32)
    iota = lax.broadcasted_iota(jnp.int32, (TM, KC), 1).astype(jnp.float32)
    for j in range(NKC):
        c = c_ref[pl.ds(j * KC, KC), :]
        cn = cn_ref[0, pl.ds(j * KC, KC)].reshape(1, KC)
        m = lax.dot_general(e, c, (((1,), (1,)), ((), ())),
                            preferred_element_type=jnp.float32)
        d2 = (a - 2.0 * m) + cn
        tmin = jnp.min(d2, axis=1)
        tidx = jnp.min(jnp.where(d2 == tmin[:, None], iota, float(KCB)), axis=1)
        upd = tmin < best
        bidx = jnp.where(upd, tidx + float(j * KC), bidx)
        best = jnp.where(upd, tmin, best)
    idx_ref[0, 0, :] = bidx.astype(jnp.int32)


def _dist_argmin(encoding, codebook):
    return pl.pallas_call(
        _dist_argmin_body,
        grid=(NT,),
        in_specs=[
            pl.BlockSpec((TM, D), lambda i: (i, 0)),
            pl.BlockSpec((KCB, D), lambda i: (0, 0)),
        ],
        out_specs=pl.BlockSpec((1, 1, TM), lambda i: (i, 0, 0)),
        out_shape=jax.ShapeDtypeStruct((NT, 1, TM), jnp.int32),
        scratch_shapes=[pltpu.VMEM((1, KCB), jnp.float32)],
        compiler_params=pltpu.CompilerParams(
            dimension_semantics=("arbitrary",)),
    )(encoding, codebook)


def _sc_gather(codebook, idx):
    idx3 = idx.reshape(NW, NCHUNK, CH)
    mesh = plsc.VectorSubcoreMesh(core_axis_name="c", subcore_axis_name="s")

    @functools.partial(
        pl.kernel,
        out_type=jax.ShapeDtypeStruct((NTOK, D), jnp.float32),
        mesh=mesh,
        scratch_types=[
            pltpu.VMEM((NCHUNK, CH), jnp.int32),
            pltpu.VMEM((CH, D), jnp.float32),
            pltpu.VMEM((CH, D), jnp.float32),
            pltpu.SemaphoreType.DMA,
            pltpu.SemaphoreType.DMA,
        ],
    )
    def k(cb_hbm, idx_hbm, out_hbm, idx_v, buf0, buf1, sem0, sem1):
        wid = lax.axis_index("s") * NC + lax.axis_index("c")
        base = wid * BPW
        pltpu.sync_copy(idx_hbm.at[wid], idx_v)
        bufs = (buf0, buf1)
        sems = (sem0, sem1)
        cp = pltpu.async_copy(cb_hbm.at[idx_v.at[0]], bufs[0], sems[0])
        for ci in range(NCHUNK):
            nxt = None
            if ci + 1 < NCHUNK:
                nxt = pltpu.async_copy(
                    cb_hbm.at[idx_v.at[ci + 1]], bufs[(ci + 1) % 2],
                    sems[(ci + 1) % 2])
            cp.wait()
            pltpu.sync_copy(bufs[ci % 2], out_hbm.at[pl.ds(base + ci * CH, CH)])
            cp = nxt

    return k(codebook, idx3)


def _st_loss_body(e_ref, q_ref, qst_ref, part_ref):
    i = pl.program_id(0)
    e = e_ref[...]
    q = q_ref[...]
    d = q - e
    qst_ref[...] = e + d

    @pl.when(i == 0)
    def _():
        part_ref[...] = jnp.zeros_like(part_ref)

    part_ref[0, 0, :] = part_ref[0, 0, :] + jnp.sum(d * d, axis=0)


def _st_loss(encoding, q):
    return pl.pallas_call(
        _st_loss_body,
        grid=(NT,),
        in_specs=[
            pl.BlockSpec((TM, D), lambda i: (i, 0)),
            pl.BlockSpec((TM, D), lambda i: (i, 0)),
        ],
        out_specs=[
            pl.BlockSpec((TM, D), lambda i: (i, 0)),
            pl.BlockSpec((1, 1, D), lambda i: (0, 0, 0)),
        ],
        out_shape=[
            jax.ShapeDtypeStruct((NTOK, D), jnp.float32),
            jax.ShapeDtypeStruct((1, 1, D), jnp.float32),
        ],
        compiler_params=pltpu.CompilerParams(
            dimension_semantics=("arbitrary",)),
    )(encoding, q)


def kernel(encoding, codebook):
    idx = _dist_argmin(encoding, codebook).reshape(NTOK)
    q = _sc_gather(codebook, idx)
    qst, parts = _st_loss(encoding, q)
    mse = jnp.sum(parts) / (NTOK * D)
    commitment_loss = mse
    embedding_loss = mse
    vq_loss = commitment_loss * BETA + embedding_loss
    return (idx.reshape(NTOK, 1), qst, vq_loss, embedding_loss, commitment_loss)


# speedup vs baseline: 1.6825x; 1.1182x over previous
import functools

import jax
import jax.numpy as jnp
from jax import lax
from jax.experimental import pallas as pl
from jax.experimental.pallas import tpu as pltpu
from jax.experimental.pallas import tpu_sc as plsc

KCB = 8192
D = 256
NTOK = 16384
BETA = 0.25

TM = 256
NT = NTOK // TM
KC = 1024
NKC = KCB // KC

NC, NS, L = 2, 16, 16
NW = NC * NS
BPW = NTOK // NW
CH = 128
NCHUNK = BPW // CH


def _dist_argmin_body(e_ref, c_ref, idx_ref, cn_ref):
    i = pl.program_id(0)

    @pl.when(i == 0)
    def _():
        for j in range(NKC):
            c = c_ref[pl.ds(j * KC, KC), :]
            cn_ref[0, pl.ds(j * KC, KC)] = jnp.sum(c * c, axis=1)

    e = e_ref[...]
    a = jnp.sum(e * e, axis=1, keepdims=True)
    run_val = jnp.full((TM, 128), jnp.inf, dtype=jnp.float32)
    run_slab = jnp.zeros((TM, 128), dtype=jnp.float32)
    for j in range(NKC):
        c = c_ref[pl.ds(j * KC, KC), :]
        cn = cn_ref[0, pl.ds(j * KC, KC)].reshape(1, KC)
        m = lax.dot_general(e, c, (((1,), (1,)), ((), ())),
                            preferred_element_type=jnp.float32)
        d2 = (a - 2.0 * m) + cn
        for s in range(KC // 128):
            slab = d2[:, s * 128:(s + 1) * 128]
            upd = slab < run_val
            run_val = jnp.where(upd, slab, run_val)
            run_slab = jnp.where(upd, float(j * (KC // 128) + s), run_slab)
    lane = lax.broadcasted_iota(jnp.int32, (TM, 128), 1).astype(jnp.float32)
    run_idx = run_slab * 128.0 + lane
    gmin = jnp.min(run_val, axis=1)
    bidx = jnp.min(jnp.where(run_val == gmin[:, None], run_idx, float(KCB)),
                   axis=1)
    idx_ref[0, 0, :] = bidx.astype(jnp.int32)


def _dist_argmin(encoding, codebook):
    return pl.pallas_call(
        _dist_argmin_body,
        grid=(NT,),
        in_specs=[
            pl.BlockSpec((TM, D), lambda i: (i, 0)),
            pl.BlockSpec((KCB, D), lambda i: (0, 0)),
        ],
        out_specs=pl.BlockSpec((1, 1, TM), lambda i: (i, 0, 0)),
        out_shape=jax.ShapeDtypeStruct((NT, 1, TM), jnp.int32),
        scratch_shapes=[pltpu.VMEM((1, KCB), jnp.float32)],
        compiler_params=pltpu.CompilerParams(
            dimension_semantics=("arbitrary",)),
    )(encoding, codebook)


def _sc_gather(codebook, idx):
    idx3 = idx.reshape(NW, NCHUNK, CH)
    mesh = plsc.VectorSubcoreMesh(core_axis_name="c", subcore_axis_name="s")

    @functools.partial(
        pl.kernel,
        out_type=jax.ShapeDtypeStruct((NTOK, D), jnp.float32),
        mesh=mesh,
        scratch_types=[
            pltpu.VMEM((NCHUNK, CH), jnp.int32),
            pltpu.VMEM((CH, D), jnp.float32),
            pltpu.VMEM((CH, D), jnp.float32),
            pltpu.SemaphoreType.DMA,
            pltpu.SemaphoreType.DMA,
        ],
    )
    def k(cb_hbm, idx_hbm, out_hbm, idx_v, buf0, buf1, sem0, sem1):
        wid = lax.axis_index("s") * NC + lax.axis_index("c")
        base = wid * BPW
        pltpu.sync_copy(idx_hbm.at[wid], idx_v)
        bufs = (buf0, buf1)
        sems = (sem0, sem1)
        cp = pltpu.async_copy(cb_hbm.at[idx_v.at[0]], bufs[0], sems[0])
        for ci in range(NCHUNK):
            nxt = None
            if ci + 1 < NCHUNK:
                nxt = pltpu.async_copy(
                    cb_hbm.at[idx_v.at[ci + 1]], bufs[(ci + 1) % 2],
                    sems[(ci + 1) % 2])
            cp.wait()
            pltpu.sync_copy(bufs[ci % 2], out_hbm.at[pl.ds(base + ci * CH, CH)])
            cp = nxt

    return k(codebook, idx3)


def _st_loss_body(e_ref, q_ref, qst_ref, part_ref):
    i = pl.program_id(0)
    e = e_ref[...]
    q = q_ref[...]
    d = q - e
    qst_ref[...] = e + d

    @pl.when(i == 0)
    def _():
        part_ref[...] = jnp.zeros_like(part_ref)

    part_ref[0, 0, :] = part_ref[0, 0, :] + jnp.sum(d * d, axis=0)


TL = 1024
NTL = NTOK // TL


def _st_loss(encoding, q):
    return pl.pallas_call(
        _st_loss_body,
        grid=(NTL,),
        in_specs=[
            pl.BlockSpec((TL, D), lambda i: (i, 0)),
            pl.BlockSpec((TL, D), lambda i: (i, 0)),
        ],
        out_specs=[
            pl.BlockSpec((TL, D), lambda i: (i, 0)),
            pl.BlockSpec((1, 1, D), lambda i: (0, 0, 0)),
        ],
        out_shape=[
            jax.ShapeDtypeStruct((NTOK, D), jnp.float32),
            jax.ShapeDtypeStruct((1, 1, D), jnp.float32),
        ],
        compiler_params=pltpu.CompilerParams(
            dimension_semantics=("arbitrary",)),
    )(encoding, q)


def kernel(encoding, codebook):
    idx = _dist_argmin(encoding, codebook).reshape(NTOK)
    q = _sc_gather(codebook, idx)
    qst, parts = _st_loss(encoding, q)
    mse = jnp.sum(parts) / (NTOK * D)
    commitment_loss = mse
    embedding_loss = mse
    vq_loss = commitment_loss * BETA + embedding_loss
    return (idx.reshape(NTOK, 1), qst, vq_loss, embedding_loss, commitment_loss)


# speedup vs baseline: 1.8351x; 1.0907x over previous
import functools

import jax
import jax.numpy as jnp
from jax import lax
from jax.experimental import pallas as pl
from jax.experimental.pallas import tpu as pltpu
from jax.experimental.pallas import tpu_sc as plsc

KCB = 8192
D = 256
NTOK = 16384
BETA = 0.25

TM = 1024
NT = NTOK // TM
KC = 1024
NKC = KCB // KC

NC, NS, L = 2, 16, 16
NW = NC * NS
BPW = NTOK // NW
CH = 128
NCHUNK = BPW // CH


def _dist_argmin_body(e_ref, c_ref, idx_ref, cn_ref):
    i = pl.program_id(0)

    @pl.when(i == 0)
    def _():
        for j in range(NKC):
            c = c_ref[pl.ds(j * KC, KC), :]
            cn_ref[0, pl.ds(j * KC, KC)] = jnp.sum(c * c, axis=1)

    e = e_ref[...]
    a = jnp.sum(e * e, axis=1, keepdims=True)
    run_val = jnp.full((TM, 128), jnp.inf, dtype=jnp.float32)
    run_slab = jnp.zeros((TM, 128), dtype=jnp.float32)
    for j in range(NKC):
        c = c_ref[pl.ds(j * KC, KC), :]
        cn = cn_ref[0, pl.ds(j * KC, KC)].reshape(1, KC)
        m = lax.dot_general(e, c, (((1,), (1,)), ((), ())),
                            preferred_element_type=jnp.float32)
        for s in range(KC // 128):
            ms = m[:, s * 128:(s + 1) * 128]
            cns = cn[:, s * 128:(s + 1) * 128]
            slab = (a - 2.0 * ms) + cns
            upd = slab < run_val
            run_val = jnp.where(upd, slab, run_val)
            run_slab = jnp.where(upd, float(j * (KC // 128) + s), run_slab)
    lane = lax.broadcasted_iota(jnp.int32, (TM, 128), 1).astype(jnp.float32)
    run_idx = run_slab * 128.0 + lane
    gmin = jnp.min(run_val, axis=1)
    bidx = jnp.min(jnp.where(run_val == gmin[:, None], run_idx, float(KCB)),
                   axis=1)
    idx_ref[0, 0, :] = bidx.astype(jnp.int32)


def _dist_argmin(encoding, codebook):
    return pl.pallas_call(
        _dist_argmin_body,
        grid=(NT,),
        in_specs=[
            pl.BlockSpec((TM, D), lambda i: (i, 0)),
            pl.BlockSpec((KCB, D), lambda i: (0, 0)),
        ],
        out_specs=pl.BlockSpec((1, 1, TM), lambda i: (i, 0, 0)),
        out_shape=jax.ShapeDtypeStruct((NT, 1, TM), jnp.int32),
        scratch_shapes=[pltpu.VMEM((1, KCB), jnp.float32)],
        compiler_params=pltpu.CompilerParams(
            dimension_semantics=("arbitrary",)),
    )(encoding, codebook)


def _sc_gather(codebook, idx):
    idx3 = idx.reshape(NW, NCHUNK, CH)
    mesh = plsc.VectorSubcoreMesh(core_axis_name="c", subcore_axis_name="s")

    @functools.partial(
        pl.kernel,
        out_type=jax.ShapeDtypeStruct((NTOK, D), jnp.float32),
        mesh=mesh,
        scratch_types=[
            pltpu.VMEM((NCHUNK, CH), jnp.int32),
            pltpu.VMEM((CH, D), jnp.float32),
            pltpu.VMEM((CH, D), jnp.float32),
            pltpu.SemaphoreType.DMA,
            pltpu.SemaphoreType.DMA,
        ],
    )
    def k(cb_hbm, idx_hbm, out_hbm, idx_v, buf0, buf1, sem0, sem1):
        wid = lax.axis_index("s") * NC + lax.axis_index("c")
        base = wid * BPW
        pltpu.sync_copy(idx_hbm.at[wid], idx_v)
        bufs = (buf0, buf1)
        sems = (sem0, sem1)
        cp = pltpu.async_copy(cb_hbm.at[idx_v.at[0]], bufs[0], sems[0])
        for ci in range(NCHUNK):
            nxt = None
            if ci + 1 < NCHUNK:
                nxt = pltpu.async_copy(
                    cb_hbm.at[idx_v.at[ci + 1]], bufs[(ci + 1) % 2],
                    sems[(ci + 1) % 2])
            cp.wait()
            pltpu.sync_copy(bufs[ci % 2], out_hbm.at[pl.ds(base + ci * CH, CH)])
            cp = nxt

    return k(codebook, idx3)


def _st_loss_body(e_ref, q_ref, qst_ref, part_ref):
    i = pl.program_id(0)
    e = e_ref[...]
    q = q_ref[...]
    d = q - e
    qst_ref[...] = e + d

    @pl.when(i == 0)
    def _():
        part_ref[...] = jnp.zeros_like(part_ref)

    part_ref[0, 0, :] = part_ref[0, 0, :] + jnp.sum(d * d, axis=0)


TL = 1024
NTL = NTOK // TL


def _st_loss(encoding, q):
    return pl.pallas_call(
        _st_loss_body,
        grid=(NTL,),
        in_specs=[
            pl.BlockSpec((TL, D), lambda i: (i, 0)),
            pl.BlockSpec((TL, D), lambda i: (i, 0)),
        ],
        out_specs=[
            pl.BlockSpec((TL, D), lambda i: (i, 0)),
            pl.BlockSpec((1, 1, D), lambda i: (0, 0, 0)),
        ],
        out_shape=[
            jax.ShapeDtypeStruct((NTOK, D), jnp.float32),
            jax.ShapeDtypeStruct((1, 1, D), jnp.float32),
        ],
        compiler_params=pltpu.CompilerParams(
            dimension_semantics=("arbitrary",)),
    )(encoding, q)


def kernel(encoding, codebook):
    idx = _dist_argmin(encoding, codebook).reshape(NTOK)
    q = _sc_gather(codebook, idx)
    qst, parts = _st_loss(encoding, q)
    mse = jnp.sum(parts) / (NTOK * D)
    commitment_loss = mse
    embedding_loss = mse
    vq_loss = commitment_loss * BETA + embedding_loss
    return (idx.reshape(NTOK, 1), qst, vq_loss, embedding_loss, commitment_loss)


# speedup vs baseline: 1.9055x; 1.0384x over previous
import functools

import jax
import jax.numpy as jnp
from jax import lax
from jax.experimental import pallas as pl
from jax.experimental.pallas import tpu as pltpu
from jax.experimental.pallas import tpu_sc as plsc

KCB = 8192
D = 256
NTOK = 16384
BETA = 0.25

TM = 1024
NT = NTOK // TM
KC = 1024
NKC = KCB // KC

NC, NS, L = 2, 16, 16
NW = NC * NS
BPW = NTOK // NW
CH = 128
NCHUNK = BPW // CH


def _dist_argmin_body(e_ref, c_ref, idx_ref, cn_ref):
    i = pl.program_id(0)

    @pl.when(i == 0)
    def _():
        for j in range(NKC):
            c = c_ref[pl.ds(j * KC, KC), :]
            cn_ref[0, pl.ds(j * KC, KC)] = jnp.sum(c * c, axis=1)

    e = e_ref[...]
    a = jnp.sum(e * e, axis=1, keepdims=True)
    run_val = jnp.full((TM, 128), jnp.inf, dtype=jnp.float32)
    run_slab = jnp.zeros((TM, 128), dtype=jnp.float32)
    for j in range(NKC):
        c = c_ref[pl.ds(j * KC, KC), :]
        cn = cn_ref[0, pl.ds(j * KC, KC)].reshape(1, KC)
        m = lax.dot_general(e, c, (((1,), (1,)), ((), ())),
                            preferred_element_type=jnp.float32)
        d = []
        for s in range(KC // 128):
            ms = m[:, s * 128:(s + 1) * 128]
            cns = cn[:, s * 128:(s + 1) * 128]
            d.append((a - 2.0 * ms) + cns)
        ids = [jnp.float32(s) for s in range(KC // 128)]
        while len(d) > 1:
            nv, ni = [], []
            for p in range(0, len(d), 2):
                lt = d[p + 1] < d[p]
                nv.append(jnp.where(lt, d[p + 1], d[p]))
                ni.append(jnp.where(lt, ids[p + 1], ids[p]))
            d, ids = nv, ni
        upd = d[0] < run_val
        run_val = jnp.where(upd, d[0], run_val)
        run_slab = jnp.where(upd, ids[0] + float(j * (KC // 128)), run_slab)
    lane = lax.broadcasted_iota(jnp.int32, (TM, 128), 1).astype(jnp.float32)
    run_idx = run_slab * 128.0 + lane
    gmin = jnp.min(run_val, axis=1)
    bidx = jnp.min(jnp.where(run_val == gmin[:, None], run_idx, float(KCB)),
                   axis=1)
    idx_ref[0, 0, :] = bidx.astype(jnp.int32)


def _dist_argmin(encoding, codebook):
    return pl.pallas_call(
        _dist_argmin_body,
        grid=(NT,),
        in_specs=[
            pl.BlockSpec((TM, D), lambda i: (i, 0)),
            pl.BlockSpec((KCB, D), lambda i: (0, 0)),
        ],
        out_specs=pl.BlockSpec((1, 1, TM), lambda i: (i, 0, 0)),
        out_shape=jax.ShapeDtypeStruct((NT, 1, TM), jnp.int32),
        scratch_shapes=[pltpu.VMEM((1, KCB), jnp.float32)],
        compiler_params=pltpu.CompilerParams(
            dimension_semantics=("arbitrary",)),
    )(encoding, codebook)


def _sc_gather(codebook, idx):
    idx3 = idx.reshape(NW, NCHUNK, CH)
    mesh = plsc.VectorSubcoreMesh(core_axis_name="c", subcore_axis_name="s")

    @functools.partial(
        pl.kernel,
        out_type=jax.ShapeDtypeStruct((NTOK, D), jnp.float32),
        mesh=mesh,
        scratch_types=[
            pltpu.VMEM((NCHUNK, CH), jnp.int32),
            pltpu.VMEM((CH, D), jnp.float32),
            pltpu.VMEM((CH, D), jnp.float32),
            pltpu.SemaphoreType.DMA,
            pltpu.SemaphoreType.DMA,
        ],
    )
    def k(cb_hbm, idx_hbm, out_hbm, idx_v, buf0, buf1, sem0, sem1):
        wid = lax.axis_index("s") * NC + lax.axis_index("c")
        base = wid * BPW
        pltpu.sync_copy(idx_hbm.at[wid], idx_v)
        bufs = (buf0, buf1)
        sems = (sem0, sem1)
        cp = pltpu.async_copy(cb_hbm.at[idx_v.at[0]], bufs[0], sems[0])
        for ci in range(NCHUNK):
            nxt = None
            if ci + 1 < NCHUNK:
                nxt = pltpu.async_copy(
                    cb_hbm.at[idx_v.at[ci + 1]], bufs[(ci + 1) % 2],
                    sems[(ci + 1) % 2])
            cp.wait()
            pltpu.sync_copy(bufs[ci % 2], out_hbm.at[pl.ds(base + ci * CH, CH)])
            cp = nxt

    return k(codebook, idx3)


def _st_loss_body(e_ref, q_ref, qst_ref, part_ref):
    i = pl.program_id(0)
    e = e_ref[...]
    q = q_ref[...]
    d = q - e
    qst_ref[...] = e + d

    @pl.when(i == 0)
    def _():
        part_ref[...] = jnp.zeros_like(part_ref)

    part_ref[0, 0, :] = part_ref[0, 0, :] + jnp.sum(d * d, axis=0)


TL = 1024
NTL = NTOK // TL


def _st_loss(encoding, q):
    return pl.pallas_call(
        _st_loss_body,
        grid=(NTL,),
        in_specs=[
            pl.BlockSpec((TL, D), lambda i: (i, 0)),
            pl.BlockSpec((TL, D), lambda i: (i, 0)),
        ],
        out_specs=[
            pl.BlockSpec((TL, D), lambda i: (i, 0)),
            pl.BlockSpec((1, 1, D), lambda i: (0, 0, 0)),
        ],
        out_shape=[
            jax.ShapeDtypeStruct((NTOK, D), jnp.float32),
            jax.ShapeDtypeStruct((1, 1, D), jnp.float32),
        ],
        compiler_params=pltpu.CompilerParams(
            dimension_semantics=("arbitrary",)),
    )(encoding, q)


def kernel(encoding, codebook):
    idx = _dist_argmin(encoding, codebook).reshape(NTOK)
    q = _sc_gather(codebook, idx)
    qst, parts = _st_loss(encoding, q)
    mse = jnp.sum(parts) / (NTOK * D)
    commitment_loss = mse
    embedding_loss = mse
    vq_loss = commitment_loss * BETA + embedding_loss
    return (idx.reshape(NTOK, 1), qst, vq_loss, embedding_loss, commitment_loss)


# speedup vs baseline: 1.9339x; 1.0149x over previous
import functools

import jax
import jax.numpy as jnp
from jax import lax
from jax.experimental import pallas as pl
from jax.experimental.pallas import tpu as pltpu
from jax.experimental.pallas import tpu_sc as plsc

KCB = 8192
D = 256
NTOK = 16384
BETA = 0.25

TM = 1024
NT = NTOK // TM
KC = 1024
NKC = KCB // KC

NC, NS, L = 2, 16, 16
NW = NC * NS
BPW = NTOK // NW
CH = 128
NCHUNK = BPW // CH


def _dist_argmin_body(e_ref, c_ref, idx_ref, cn_ref):
    i = pl.program_id(0)

    @pl.when(i == 0)
    def _():
        for j in range(NKC):
            c = c_ref[pl.ds(j * KC, KC), :]
            cn_ref[0, pl.ds(j * KC, KC)] = jnp.sum(c * c, axis=1)

    e = e_ref[...]
    a = jnp.sum(e * e, axis=1, keepdims=True)
    run_val = jnp.full((TM, 128), jnp.inf, dtype=jnp.float32)
    run_slab = jnp.zeros((TM, 128), dtype=jnp.float32)
    for j in range(NKC):
        c = c_ref[pl.ds(j * KC, KC), :]
        cn = cn_ref[0, pl.ds(j * KC, KC)].reshape(1, KC)
        m = lax.dot_general(e, c, (((1,), (1,)), ((), ())),
                            preferred_element_type=jnp.float32)
        d = []
        for s in range(KC // 128):
            ms = m[:, s * 128:(s + 1) * 128]
            cns = cn[:, s * 128:(s + 1) * 128]
            d.append((a - 2.0 * ms) + cns)
        ids = [jnp.float32(s) for s in range(KC // 128)]
        while len(d) > 1:
            nv, ni = [], []
            for p in range(0, len(d), 2):
                lt = d[p + 1] < d[p]
                nv.append(jnp.where(lt, d[p + 1], d[p]))
                ni.append(jnp.where(lt, ids[p + 1], ids[p]))
            d, ids = nv, ni
        upd = d[0] < run_val
        run_val = jnp.where(upd, d[0], run_val)
        run_slab = jnp.where(upd, ids[0] + float(j * (KC // 128)), run_slab)
    lane = lax.broadcasted_iota(jnp.int32, (TM, 128), 1).astype(jnp.float32)
    run_idx = run_slab * 128.0 + lane
    gmin = jnp.min(run_val, axis=1)
    bidx = jnp.min(jnp.where(run_val == gmin[:, None], run_idx, float(KCB)),
                   axis=1)
    idx_ref[0, 0, :] = bidx.astype(jnp.int32)


def _dist_argmin(encoding, codebook):
    return pl.pallas_call(
        _dist_argmin_body,
        grid=(NT,),
        in_specs=[
            pl.BlockSpec((TM, D), lambda i: (i, 0)),
            pl.BlockSpec((KCB, D), lambda i: (0, 0)),
        ],
        out_specs=pl.BlockSpec((1, 1, TM), lambda i: (i, 0, 0)),
        out_shape=jax.ShapeDtypeStruct((NT, 1, TM), jnp.int32),
        scratch_shapes=[pltpu.VMEM((1, KCB), jnp.float32)],
        compiler_params=pltpu.CompilerParams(
            dimension_semantics=("arbitrary",)),
    )(encoding, codebook)


def _sc_gather(codebook, idx):
    idx3 = idx.reshape(NW, NCHUNK, CH)
    mesh = plsc.VectorSubcoreMesh(core_axis_name="c", subcore_axis_name="s")

    @functools.partial(
        pl.kernel,
        out_type=jax.ShapeDtypeStruct((NTOK, D), jnp.float32),
        mesh=mesh,
        scratch_types=[
            pltpu.VMEM((NCHUNK, CH), jnp.int32),
            pltpu.VMEM((CH, D), jnp.float32),
            pltpu.VMEM((CH, D), jnp.float32),
            pltpu.SemaphoreType.DMA,
            pltpu.SemaphoreType.DMA,
        ],
    )
    def k(cb_hbm, idx_hbm, out_hbm, idx_v, buf0, buf1, sem0, sem1):
        wid = lax.axis_index("s") * NC + lax.axis_index("c")
        base = wid * BPW
        pltpu.sync_copy(idx_hbm.at[wid], idx_v)
        bufs = (buf0, buf1)
        sems = (sem0, sem1)
        cp = pltpu.async_copy(cb_hbm.at[idx_v.at[0]], bufs[0], sems[0])
        for ci in range(NCHUNK):
            nxt = None
            if ci + 1 < NCHUNK:
                nxt = pltpu.async_copy(
                    cb_hbm.at[idx_v.at[ci + 1]], bufs[(ci + 1) % 2],
                    sems[(ci + 1) % 2])
            cp.wait()
            pltpu.sync_copy(bufs[ci % 2], out_hbm.at[pl.ds(base + ci * CH, CH)])
            cp = nxt

    return k(codebook, idx3)


def _st_loss_body(e_ref, q_ref, qst_ref, part_ref):
    i = pl.program_id(0)
    e = e_ref[...]
    q = q_ref[...]
    d = q - e
    qst_ref[...] = e + d

    @pl.when(i == 0)
    def _():
        part_ref[...] = jnp.zeros_like(part_ref)

    part_ref[0, 0, :] = part_ref[0, 0, :] + jnp.sum(d * d, axis=0)


TL = 2048
NTL = NTOK // TL


def _st_loss(encoding, q):
    return pl.pallas_call(
        _st_loss_body,
        grid=(NTL,),
        in_specs=[
            pl.BlockSpec((TL, D), lambda i: (i, 0)),
            pl.BlockSpec((TL, D), lambda i: (i, 0)),
        ],
        out_specs=[
            pl.BlockSpec((TL, D), lambda i: (i, 0)),
            pl.BlockSpec((1, 1, D), lambda i: (0, 0, 0)),
        ],
        out_shape=[
            jax.ShapeDtypeStruct((NTOK, D), jnp.float32),
            jax.ShapeDtypeStruct((1, 1, D), jnp.float32),
        ],
        compiler_params=pltpu.CompilerParams(
            dimension_semantics=("arbitrary",)),
    )(encoding, q)


def kernel(encoding, codebook):
    idx = _dist_argmin(encoding, codebook).reshape(NTOK)
    q = _sc_gather(codebook, idx)
    qst, parts = _st_loss(encoding, q)
    mse = jnp.sum(parts) / (NTOK * D)
    commitment_loss = mse
    embedding_loss = mse
    vq_loss = commitment_loss * BETA + embedding_loss
    return (idx.reshape(NTOK, 1), qst, vq_loss, embedding_loss, commitment_loss)


# speedup vs baseline: 1.9367x; 1.0015x over previous
import functools

import jax
import jax.numpy as jnp
from jax import lax
from jax.experimental import pallas as pl
from jax.experimental.pallas import tpu as pltpu
from jax.experimental.pallas import tpu_sc as plsc

KCB = 8192
D = 256
NTOK = 16384
BETA = 0.25

TM = 1024
NT = NTOK // TM
KC = 1024
NKC = KCB // KC

NC, NS, L = 2, 16, 16
NW = NC * NS
BPW = NTOK // NW
CH = 128
NCHUNK = BPW // CH


def _dist_argmin_body(e_ref, c_ref, idx_ref, cn_ref):
    i = pl.program_id(0)

    @pl.when(i == 0)
    def _():
        for j in range(NKC):
            c = c_ref[pl.ds(j * KC, KC), :]
            cn_ref[0, pl.ds(j * KC, KC)] = jnp.sum(c * c, axis=1)

    e = e_ref[...]
    a = jnp.sum(e * e, axis=1, keepdims=True)
    run_val = jnp.full((TM, 128), jnp.inf, dtype=jnp.float32)
    run_slab = jnp.zeros((TM, 128), dtype=jnp.float32)
    for j in range(NKC):
        c = c_ref[pl.ds(j * KC, KC), :]
        cn = cn_ref[0, pl.ds(j * KC, KC)].reshape(1, KC)
        m = lax.dot_general(e, c, (((1,), (1,)), ((), ())),
                            preferred_element_type=jnp.float32)
        d = []
        for s in range(KC // 128):
            ms = m[:, s * 128:(s + 1) * 128]
            cns = cn[:, s * 128:(s + 1) * 128]
            d.append((a - 2.0 * ms) + cns)
        ids = [jnp.float32(s) for s in range(KC // 128)]
        while len(d) > 1:
            nv, ni = [], []
            for p in range(0, len(d), 2):
                lt = d[p + 1] < d[p]
                nv.append(jnp.where(lt, d[p + 1], d[p]))
                ni.append(jnp.where(lt, ids[p + 1], ids[p]))
            d, ids = nv, ni
        upd = d[0] < run_val
        run_val = jnp.where(upd, d[0], run_val)
        run_slab = jnp.where(upd, ids[0] + float(j * (KC // 128)), run_slab)
    lane = lax.broadcasted_iota(jnp.int32, (TM, 128), 1).astype(jnp.float32)
    run_idx = run_slab * 128.0 + lane
    gmin = jnp.min(run_val, axis=1)
    bidx = jnp.min(jnp.where(run_val == gmin[:, None], run_idx, float(KCB)),
                   axis=1)
    idx_ref[0, 0, :] = bidx.astype(jnp.int32)


def _dist_argmin(encoding, codebook):
    return pl.pallas_call(
        _dist_argmin_body,
        grid=(NT,),
        in_specs=[
            pl.BlockSpec((TM, D), lambda i: (i, 0)),
            pl.BlockSpec((KCB, D), lambda i: (0, 0)),
        ],
        out_specs=pl.BlockSpec((1, 1, TM), lambda i: (i, 0, 0)),
        out_shape=jax.ShapeDtypeStruct((NT, 1, TM), jnp.int32),
        scratch_shapes=[pltpu.VMEM((1, KCB), jnp.float32)],
        compiler_params=pltpu.CompilerParams(
            dimension_semantics=("arbitrary",)),
    )(encoding, codebook)


def _sc_quantize(codebook, idx, encoding):
    idx3 = idx.reshape(NW, NCHUNK, CH)
    mesh = plsc.VectorSubcoreMesh(core_axis_name="c", subcore_axis_name="s")

    @functools.partial(
        pl.kernel,
        out_type=[
            jax.ShapeDtypeStruct((NTOK, D), jnp.float32),
            jax.ShapeDtypeStruct((NW, L), jnp.float32),
        ],
        mesh=mesh,
        scratch_types=[
            pltpu.VMEM((NCHUNK, CH), jnp.int32),
            pltpu.VMEM((CH, D), jnp.float32),
            pltpu.VMEM((CH, D), jnp.float32),
            pltpu.VMEM((CH, D), jnp.float32),
            pltpu.VMEM((L,), jnp.float32),
            pltpu.SemaphoreType.DMA,
            pltpu.SemaphoreType.DMA,
        ],
    )
    def k(cb_hbm, idx_hbm, e_hbm, out_hbm, part_hbm,
          idx_v, q0, q1, e_v, acc_v, sem0, sem1):
        wid = lax.axis_index("s") * NC + lax.axis_index("c")
        base = wid * BPW
        pltpu.sync_copy(idx_hbm.at[wid], idx_v)
        qb = (q0, q1)
        sems = (sem0, sem1)
        cp = pltpu.async_copy(cb_hbm.at[idx_v.at[0]], qb[0], sems[0])
        acc = jnp.zeros((L,), jnp.float32)
        for ci in range(NCHUNK):
            nxt = None
            if ci + 1 < NCHUNK:
                nxt = pltpu.async_copy(
                    cb_hbm.at[idx_v.at[ci + 1]], qb[(ci + 1) % 2],
                    sems[(ci + 1) % 2])
            row0 = base + ci * CH
            pltpu.sync_copy(e_hbm.at[pl.ds(row0, CH)], e_v)
            cp.wait()
            q = qb[ci % 2]

            def body(r, acc, q=q):
                for t in range(D // L):
                    sl = pl.ds(t * L, L)
                    qv = q[r, sl]
                    ev = e_v[r, sl]
                    dv = qv - ev
                    acc = acc + dv * dv
                    q[r, sl] = ev + dv
                return acc

            acc = lax.fori_loop(0, CH, body, acc)
            pltpu.sync_copy(q, out_hbm.at[pl.ds(row0, CH)])
            cp = nxt
        acc_v[...] = acc
        pltpu.sync_copy(acc_v, part_hbm.at[wid])

    return k(codebook, idx3, encoding)


def kernel(encoding, codebook):
    idx = _dist_argmin(encoding, codebook).reshape(NTOK)
    qst, parts = _sc_quantize(codebook, idx, encoding)
    mse = jnp.sum(parts) / (NTOK * D)
    commitment_loss = mse
    embedding_loss = mse
    vq_loss = commitment_loss * BETA + embedding_loss
    return (idx.reshape(NTOK, 1), qst, vq_loss, embedding_loss, commitment_loss)
